# baseline (speedup 1.0000x reference)
import functools

import jax
import jax.numpy as jnp
from jax import lax
from jax.experimental import pallas as pl
from jax.experimental.pallas import tpu as pltpu
from jax.experimental.pallas import tpu_sc as plsc

N = 10000
E = 320000
D = 128
H = 128
Hh = H // 2
R = 3
G = 64

NSC = 2
NT = 16
Nh = N // NSC

SEG = 5120
GARB = 5000
ACC_ROWS = R * SEG
TSLICE = ACC_ROWS // NT
ZCH = 48

DR = 30720
DEGSZ = 2 * DR
DGARB = R * N
DSLICE = DEGSZ // NT

EPAD = 327680
CH = 128
NCH = EPAD // CH
LMAX = 160
NCHP = NCH + LMAX
NCHK = NCH // (NSC * NT)

_f32 = jnp.float32
_i32 = jnp.int32
_HIGH = jax.lax.Precision.HIGHEST


@functools.cache
def _vmesh():
    return plsc.VectorSubcoreMesh(core_axis_name="c", subcore_axis_name="s",
                                  num_cores=NSC, num_subcores=NT)


K0B = CH
K0NB = NCH // K0B


def _block_positions(m, carry):
    mf = m.astype(_f32)
    ii = lax.broadcasted_iota(_i32, (CH, CH), 0)
    jj = lax.broadcasted_iota(_i32, (CH, CH), 1)
    lt = (ii <= jj).astype(_f32)
    sl = (ii > jj).astype(_f32)
    intra = jnp.dot(mf, lt, precision=_HIGH, preferred_element_type=_f32)
    tot_b = intra[:, CH - 1:CH]
    chunk_excl = jnp.dot(sl, tot_b, precision=_HIGH,
                         preferred_element_type=_f32)
    pos = chunk_excl + intra - mf + carry
    return pos, jnp.sum(mf)


def _k0_body(src, dst, typ, fsrc_g, fsrc_d, fdst_d, lrow, lrow1, pos, cnts,
             tot0, tot1, car0, car1):
    ph = pl.program_id(0)
    i = pl.program_id(1)
    s = src[...]
    d = dst[...]
    t = typ[...]
    real = d < N
    half = d >= Nh
    m0 = real & jnp.logical_not(half)
    m1 = real & half

    @pl.when(ph == 0)
    def _():
        @pl.when(i == 0)
        def _():
            tot0[...] = jnp.zeros((1, 1), _f32)
            tot1[...] = jnp.zeros((1, 1), _f32)
        tot0[...] += jnp.sum(m0.astype(_f32)).reshape(1, 1)
        tot1[...] += jnp.sum(m1.astype(_f32)).reshape(1, 1)

    @pl.when(ph == 1)
    def _():
        fsrc_g[...] = jnp.where(real, t * N + s, 0)
        fsrc_d[...] = jnp.where(real, t * N + s, DGARB)
        fdst_d[...] = jnp.where(real, DR + t * N + d, DGARB)
        lrow[...] = jnp.where(m0, t * SEG + d, GARB)
        lrow1[...] = jnp.where(m1, t * SEG + d - Nh, GARB)

        @pl.when(i == 0)
        def _():
            car0[...] = jnp.zeros((1, 1), _f32)
            car1[...] = jnp.zeros((1, 1), _f32)

        cin0 = car0[0, 0]
        cin1 = car1[0, 0]
        p0, t0 = _block_positions(m0, cin0)
        p1, t1 = _block_positions(m1, cin1)
        car0[...] = (cin0 + t0).reshape(1, 1)
        car1[...] = (cin1 + t1).reshape(1, 1)

        c0 = tot0[0, 0].astype(_i32)
        c1 = tot1[0, 0].astype(_i32)
        cl = (c0 + CH - 1) // CH
        ceu = cl + (c1 + CH - 1) // CH
        g1 = cl * CH - c0

        eidx = ((lax.broadcasted_iota(_i32, (K0B, CH), 0) + i * K0B) * CH
                + lax.broadcasted_iota(_i32, (K0B, CH), 1))
        k = eidx - E
        padpos = jnp.where(k < g1, c0 + k, cl * CH + c1 + (k - g1))
        pos[...] = jnp.where(m0, p0.astype(_i32),
                             jnp.where(m1, cl * CH + p1.astype(_i32),
                                       padpos))

        @pl.when(i == K0NB - 1)
        def _():
            lane = lax.broadcasted_iota(_i32, (CH,), 0)
            vals = (jnp.where(lane == 0, cl, 0)
                    + jnp.where(lane == 1, ceu - cl, 0)
                    + jnp.where(lane == 3, cl, 0))
            cnts[...] = vals.astype(_f32)


def _k0(src, dst, typ, interpret=False):
    shp = jax.ShapeDtypeStruct(src.shape, _i32)
    bspec = pl.BlockSpec((K0B, CH), lambda ph, i: (i, 0))
    return pl.pallas_call(
        _k0_body,
        grid=(2, K0NB),
        in_specs=[bspec] * 3,
        out_specs=[bspec] * 6 + [pl.BlockSpec((CH,), lambda ph, i: (0,))],
        out_shape=[shp] * 6 + [jax.ShapeDtypeStruct((CH,), _f32)],
        scratch_shapes=[pltpu.VMEM((1, 1), _f32)] * 4,
        interpret=interpret,
    )(src, dst, typ)


def _k1_body(fsrc_hbm, fdst_hbm, fg_hbm, lr_hbm, pos_hbm,
             out_hbm, outf_hbm, outl_hbm,
             av, bv, fv, lv, pv, ones_v, zero_v, degacc, sem_r):
    c = lax.axis_index("c")
    s = lax.axis_index("s")

    @pl.loop(0, CH, step=16)
    def _(i):
        ones_v[pl.ds(i, 16)] = jnp.ones((16,), _f32)
        zero_v[pl.ds(i, 16)] = jnp.zeros((16,), _f32)

    @pl.loop(0, DSLICE, step=CH)
    def _(i):
        pltpu.sync_copy(zero_v, degacc.at[pl.ds(s * DSLICE + i, CH)])

    plsc.subcore_barrier()

    base = (c * NT + s) * NCHK
    pltpu.sync_copy(fsrc_hbm.at[pl.ds(base, NCHK)], av)
    pltpu.sync_copy(fdst_hbm.at[pl.ds(base, NCHK)], bv)
    pltpu.sync_copy(fg_hbm.at[pl.ds(base, NCHK)], fv)
    pltpu.sync_copy(lr_hbm.at[pl.ds(base, NCHK)], lv)
    pltpu.sync_copy(pos_hbm.at[pl.ds(base, NCHK)], pv)

    @pl.loop(0, NCHK)
    def _(i):
        pltpu.async_copy(fv.at[i], outf_hbm.at[pv.at[i]], sem_r)
        pltpu.async_copy(lv.at[i], outl_hbm.at[pv.at[i]], sem_r)
        pltpu.sync_copy(ones_v, degacc.at[av.at[i]], add=True)
        pltpu.sync_copy(ones_v, degacc.at[bv.at[i]], add=True)

    @pl.loop(0, 2 * NCHK)
    def _(i):
        pltpu.make_async_copy(fv.at[0], outf_hbm.at[pv.at[0]], sem_r).wait()

    plsc.subcore_barrier()
    pltpu.sync_copy(degacc.at[pl.ds(s * DSLICE, DSLICE)],
                    out_hbm.at[c, pl.ds(s * DSLICE, DSLICE)])


@functools.cache
def _k1_built():
    return pl.kernel(
        _k1_body,
        out_type=[
            jax.ShapeDtypeStruct((NSC, DEGSZ), _f32),
            jax.ShapeDtypeStruct((NCHP * CH,), _i32),
            jax.ShapeDtypeStruct((NCHP * CH,), _i32),
        ],
        mesh=_vmesh(),
        scratch_types=[
            pltpu.VMEM((NCHK, CH), _i32),
            pltpu.VMEM((NCHK, CH), _i32),
            pltpu.VMEM((NCHK, CH), _i32),
            pltpu.VMEM((NCHK, CH), _i32),
            pltpu.VMEM((NCHK, CH), _i32),
            pltpu.VMEM((CH,), _f32),
            pltpu.VMEM((CH,), _f32),
            pltpu.VMEM_SHARED((DEGSZ,), _f32),
            pltpu.SemaphoreType.DMA,
        ],
    )


def _k1(fsrc_d, fdst_d, fsrc_g, lrow, pos):
    return _k1_built()(fsrc_d, fdst_d, fsrc_g, lrow, pos)


def _ksc_agg_body(ftab_hbm, fsrc_hbm, lrow_hbm, cnt_hbm, out_hbm,
                  cntv, gi, si, rows_a, rows_b, zbuf, acc, sem_a, sem_b):
    c = lax.axis_index("c")
    s = lax.axis_index("s")

    @pl.loop(0, ZCH)
    def _(r):
        @pl.loop(0, Hh, step=16)
        def _(l):
            zbuf[r, pl.ds(l, 16)] = jnp.zeros((16,), _f32)

    @pl.loop(0, TSLICE, step=ZCH)
    def _(i):
        pltpu.sync_copy(zbuf, acc.at[pl.ds(s * TSLICE + i, ZCH)])

    pltpu.sync_copy(cnt_hbm, cntv)
    base = s * LMAX
    pltpu.sync_copy(fsrc_hbm.at[pl.ds(base, LMAX)], gi.at[pl.ds(0, LMAX)])
    pltpu.sync_copy(lrow_hbm.at[pl.ds(c * NCH + base, LMAX)],
                    si.at[pl.ds(0, LMAX)])

    plsc.subcore_barrier()

    def _chunk(j, carry):
        pltpu.sync_copy(ftab_hbm.at[gi.at[j]], rows_a)
        pltpu.sync_copy(rows_a, acc.at[si.at[j]], add=True)
        return carry

    lax.fori_loop(0, LMAX, _chunk, 0)

    plsc.subcore_barrier()
    pltpu.sync_copy(acc.at[pl.ds(s * TSLICE, TSLICE)],
                    out_hbm.at[c, pl.ds(s * TSLICE, TSLICE)])


@functools.cache
def _ksc_agg_built():
    return pl.kernel(
        _ksc_agg_body,
        out_type=jax.ShapeDtypeStruct((NSC, ACC_ROWS, Hh), _f32),
        mesh=_vmesh(),
        compiler_params=pltpu.CompilerParams(use_tc_tiling_on_sc=False),
        scratch_types=[
            pltpu.VMEM((CH,), _f32),
            pltpu.VMEM((LMAX + 1, CH), _i32),
            pltpu.VMEM((LMAX + 1, CH), _i32),
            pltpu.VMEM((CH, Hh), _f32),
            pltpu.VMEM((CH, Hh), _f32),
            pltpu.VMEM((ZCH, Hh), _f32),
            pltpu.VMEM_SHARED((ACC_ROWS, Hh), _f32),
            pltpu.SemaphoreType.DMA,
            pltpu.SemaphoreType.DMA,
        ],
    )


def _ksc_agg(ftab, fsrc, lrow, cnts):
    return _ksc_agg_built()(ftab, fsrc, lrow, cnts)


def _k2_body(x, da, db, outA, outB):
    deg = da[...] + db[...]
    norm = lax.rsqrt(jnp.maximum(deg, 1.0))
    f = x[...] * norm
    outA[...] = f[:, :Hh]
    outB[...] = f[:, Hh:]


def _k2(x, degA, degB, interpret=False):
    nb = R * N // 1000
    return pl.pallas_call(
        _k2_body,
        grid=(nb,),
        in_specs=[
            pl.BlockSpec((1000, D), lambda i: (i % (N // 1000), 0)),
            pl.BlockSpec((1000, 1), lambda i: (i, 0)),
            pl.BlockSpec((1000, 1), lambda i: (i, 0)),
        ],
        out_specs=[pl.BlockSpec((1000, Hh), lambda i: (i, 0))] * 2,
        out_shape=[jax.ShapeDtypeStruct((R * N, Hh), _f32)] * 2,
        interpret=interpret,
    )(x, degA, degB)


def _agg_spec():
    return pl.BlockSpec((1, R, 1000, Hh),
                        lambda j: (j // (Nh // 1000), 0, j % (Nh // 1000), 0))


def _k4_body(aggA, aggB, dia, dib, doa, dob, W, b, outA, outB):
    agg = jnp.concatenate([aggA[0], aggB[0]], axis=-1)
    ndst = lax.rsqrt(jnp.maximum(dia[...] + dib[...], 1.0))
    h = jnp.zeros((1000, H), _f32)
    for r in range(R):
        h = h + jnp.dot(agg[r] * ndst[r], W[r], precision=_HIGH,
                        preferred_element_type=_f32)
    h = h + jnp.sum(b[...], axis=0)
    h = jnp.maximum(h, 0.0)
    nsrc = lax.rsqrt(jnp.maximum(doa[...] + dob[...], 1.0))
    f = h[None, :, :] * nsrc
    outA[...] = f[..., :Hh]
    outB[...] = f[..., Hh:]


def _k4(aggA, aggB, diA, diB, doA, doB, W, b, interpret=False):
    nb = N // 1000
    return pl.pallas_call(
        _k4_body,
        grid=(nb,),
        in_specs=[
            _agg_spec(),
            _agg_spec(),
            pl.BlockSpec((R, 1000, 1), lambda j: (0, j, 0)),
            pl.BlockSpec((R, 1000, 1), lambda j: (0, j, 0)),
            pl.BlockSpec((R, 1000, 1), lambda j: (0, j, 0)),
            pl.BlockSpec((R, 1000, 1), lambda j: (0, j, 0)),
            pl.BlockSpec((R, H, H), lambda j: (0, 0, 0)),
            pl.BlockSpec((R, 1, H), lambda j: (0, 0, 0)),
        ],
        out_specs=[pl.BlockSpec((R, 1000, Hh), lambda j: (0, j, 0))] * 2,
        out_shape=[jax.ShapeDtypeStruct((R, N, Hh), _f32)] * 2,
        interpret=interpret,
    )(aggA, aggB, diA, diB, doA, doB, W, b)


def _k6_body(aggA, aggB, dia, dib, W, b, gids, out, sums, counts):
    j = pl.program_id(0)
    nb = pl.num_programs(0)
    agg = jnp.concatenate([aggA[0], aggB[0]], axis=-1)
    ndst = lax.rsqrt(jnp.maximum(dia[...] + dib[...], 1.0))
    h = jnp.zeros((1000, H), _f32)
    for r in range(R):
        h = h + jnp.dot(agg[r] * ndst[r], W[r], precision=_HIGH,
                        preferred_element_type=_f32)
    h = h + jnp.sum(b[...], axis=0)
    giota = lax.broadcasted_iota(_i32, (1000, G), 1)
    P = (gids[...] == giota).astype(_f32)
    dn = (((0,), (0,)), ((), ()))
    psum = lax.dot_general(P, h, dn, precision=_HIGH,
                           preferred_element_type=_f32)
    pcnt = lax.dot_general(P, jnp.ones((1000, H), _f32), dn, precision=_HIGH,
                           preferred_element_type=_f32)

    @pl.when(j == 0)
    def _():
        sums[...] = psum
        counts[...] = pcnt

    @pl.when(j > 0)
    def _():
        sums[...] += psum
        counts[...] += pcnt

    @pl.when(j == nb - 1)
    def _():
        out[...] = sums[...] / jnp.maximum(counts[...], 1.0)


def _k6(aggA, aggB, diA, diB, W, b, gids, interpret=False):
    nb = N // 1000
    return pl.pallas_call(
        _k6_body,
        grid=(nb,),
        in_specs=[
            _agg_spec(),
            _agg_spec(),
            pl.BlockSpec((R, 1000, 1), lambda j: (0, j, 0)),
            pl.BlockSpec((R, 1000, 1), lambda j: (0, j, 0)),
            pl.BlockSpec((R, H, H), lambda j: (0, 0, 0)),
            pl.BlockSpec((R, 1, H), lambda j: (0, 0, 0)),
            pl.BlockSpec((1000, 1), lambda j: (j, 0)),
        ],
        out_specs=pl.BlockSpec((G, H), lambda j: (0, 0)),
        out_shape=jax.ShapeDtypeStruct((G, H), _f32),
        scratch_shapes=[pltpu.VMEM((G, H), _f32), pltpu.VMEM((G, H), _f32)],
        interpret=interpret,
    )(aggA, aggB, diA, diB, W, b, gids)


def kernel(x, edge_index, edge_type, graph_ids, W1, b1, W2, b2):
    src = edge_index[0].astype(_i32)
    dst = edge_index[1].astype(_i32)
    typ = edge_type.astype(_i32)

    pad = EPAD - E
    src_p = jnp.pad(src, (0, pad))
    dst_p = jnp.pad(dst, (0, pad), constant_values=10**8)
    typ_p = jnp.pad(typ, (0, pad))

    fsrc_g, fsrc_d, fdst_d, lrow0, lrow1, pos, cnts = _k0(
        src_p.reshape(-1, 128), dst_p.reshape(-1, 128),
        typ_p.reshape(-1, 128))

    degp, _, _ = _k1(fsrc_d, fdst_d, fsrc_g, lrow0, pos)
    fsrc2 = fsrc_g
    lrow2 = jnp.concatenate([lrow0, lrow1], axis=0)
    doA = degp[0, :R * N].reshape(R * N, 1)
    doB = degp[1, :R * N].reshape(R * N, 1)
    diA = degp[0, DR:DR + R * N].reshape(R * N, 1)
    diB = degp[1, DR:DR + R * N].reshape(R * N, 1)
    diA3 = diA.reshape(R, N, 1)
    diB3 = diB.reshape(R, N, 1)
    doA3 = doA.reshape(R, N, 1)
    doB3 = doB.reshape(R, N, 1)

    F1a, F1b = _k2(x, doA, doB)
    o1a = _ksc_agg(F1a, fsrc2, lrow2, cnts).reshape(NSC, R, SEG, Hh)
    o1b = _ksc_agg(F1b, fsrc2, lrow2, cnts).reshape(NSC, R, SEG, Hh)

    F2a, F2b = _k4(o1a, o1b, diA3, diB3, doA3, doB3, W1, b1.reshape(R, 1, H))
    o2a = _ksc_agg(F2a.reshape(R * N, Hh), fsrc2, lrow2, cnts).reshape(
        NSC, R, SEG, Hh)
    o2b = _ksc_agg(F2b.reshape(R * N, Hh), fsrc2, lrow2, cnts).reshape(
        NSC, R, SEG, Hh)

    hg = _k6(o2a, o2b, diA3, diB3, W2, b2.reshape(R, 1, H),
             graph_ids.astype(_i32).reshape(N, 1))
    return hg
